# Initial kernel scaffold; baseline (speedup 1.0000x reference)
import jax
import jax.numpy as jnp
from jax.experimental import pallas as pl


def kernel(group_ids, tech_ids, Eg, Et, Wg1, bg1, Wg2, bg2, Wg3, bg3, Wt1, bt1, Wt2, bt2, Wt3, bt3):
    raise NotImplementedError("write your pallas kernel here")


# speedup vs baseline: 39.2008x; 39.2008x over previous
import functools

import jax
import jax.numpy as jnp
from jax import lax
from jax.experimental import pallas as pl
from jax.experimental.pallas import tpu as pltpu
from jax.experimental.pallas import tpu_sc as plsc

_B = 4096
_LG = 200
_LT = 400
_V = 1000
_D = 32
_H = 256
_OUT = 64
_W = 2048
_TOFF = 1024

_NC = 2
_NS = 16
_LANES = 16
_NW = _NC * _NS
_RPW = _B // _NW
_R = 16
_NCHUNK = _RPW // _R


def _sc_histogram(gid_flat, tid_flat):
    mesh = plsc.VectorSubcoreMesh(core_axis_name="c", subcore_axis_name="s")

    @functools.partial(
        pl.kernel,
        mesh=mesh,
        compiler_params=pltpu.CompilerParams(needs_layout_passes=False),
        out_type=jax.ShapeDtypeStruct((_B * _W,), jnp.float32),
        scratch_types=[
            pltpu.VMEM((_R * _LG,), jnp.int32),
            pltpu.VMEM((_R * _LT,), jnp.int32),
            pltpu.VMEM((_R * _W,), jnp.float32),
        ],
    )
    def hist_kernel(gid_hbm, tid_hbm, out_hbm, gid_v, tid_v, hist_v):
        wid = lax.axis_index("s") * _NC + lax.axis_index("c")
        base = wid * _RPW
        ones = jnp.ones((_LANES,), jnp.float32)
        lane = lax.iota(jnp.int32, _LANES)
        tailmask = lane >= (16 - _LG % 16)

        def chunk_body(c, carry):
            row0 = base + c * _R
            pltpu.sync_copy(gid_hbm.at[pl.ds(row0 * _LG, _R * _LG)], gid_v)
            pltpu.sync_copy(tid_hbm.at[pl.ds(row0 * _LT, _R * _LT)], tid_v)

            def zero_body(j, carry2):
                for k in range(8):
                    hist_v[pl.ds(j * 128 + k * 16, 16)] = jnp.zeros(
                        (16,), jnp.float32)
                return carry2

            lax.fori_loop(0, (_R * _W) // 128, zero_body, 0)

            def row_body(r, carry2):
                gb = jnp.full((_LANES,), r * _W, jnp.int32)
                tb = jnp.full((_LANES,), r * _W + _TOFF, jnp.int32)
                for cc in range(_LG // 16):
                    g = gid_v[pl.ds(r * _LG + cc * 16, 16)]
                    plsc.addupdate_scatter(hist_v, [g + gb], ones)
                g = gid_v[pl.ds(r * _LG + (_LG - 16), 16)]
                plsc.addupdate_scatter(hist_v, [g + gb], ones, mask=tailmask)
                for cc in range(_LT // 16):
                    t = tid_v[pl.ds(r * _LT + cc * 16, 16)]
                    plsc.addupdate_scatter(hist_v, [t + tb], ones)
                return carry2

            lax.fori_loop(0, _R, row_body, 0)

            pltpu.sync_copy(hist_v, out_hbm.at[pl.ds(row0 * _W, _R * _W)])
            return carry

        lax.fori_loop(0, _NCHUNK, chunk_body, 0)

    return hist_kernel(gid_flat, tid_flat)


def _tc_towers(counts, T, Wg1, bg1, Wg2, bg2, Wg3, bg3,
               Wt1, bt1, Wt2, bt2, Wt3, bt3):
    BB = 256
    hp = lax.Precision.HIGHEST

    def body(c_ref, t_ref, wg1, bg1r, wg2, bg2r, wg3, bg3r,
             wt1, bt1r, wt2, bt2r, wt3, bt3r, out_ref):
        x = jnp.dot(c_ref[...], t_ref[...],
                    preferred_element_type=jnp.float32, precision=hp)
        g = x[:, 0:_D] / jnp.maximum(x[:, _D:_D + 1], 1.0)
        t = x[:, 64:64 + _D] / jnp.maximum(x[:, 64 + _D:64 + _D + 1], 1.0)
        hg = jnp.maximum(jnp.dot(g, wg1[...], precision=hp) + bg1r[...], 0.0)
        hg = jnp.maximum(jnp.dot(hg, wg2[...], precision=hp) + bg2r[...], 0.0)
        gv = jnp.dot(hg, wg3[...], precision=hp) + bg3r[...]
        ht = jnp.maximum(jnp.dot(t, wt1[...], precision=hp) + bt1r[...], 0.0)
        ht = jnp.maximum(jnp.dot(ht, wt2[...], precision=hp) + bt2r[...], 0.0)
        tv = jnp.dot(ht, wt3[...], precision=hp) + bt3r[...]
        out_ref[...] = jax.nn.sigmoid(jnp.sum(gv * tv, axis=1))

    full = lambda shape: pl.BlockSpec(shape, lambda i: (0,) * len(shape))
    return pl.pallas_call(
        body,
        grid=(_B // BB,),
        in_specs=[
            pl.BlockSpec((BB, _W), lambda i: (i, 0)),
            full((_W, 128)),
            full((_D, _H)), full((1, _H)),
            full((_H, _H)), full((1, _H)),
            full((_H, _OUT)), full((1, _OUT)),
            full((_D, _H)), full((1, _H)),
            full((_H, _H)), full((1, _H)),
            full((_H, _OUT)), full((1, _OUT)),
        ],
        out_specs=pl.BlockSpec((BB,), lambda i: (i,)),
        out_shape=jax.ShapeDtypeStruct((_B,), jnp.float32),
    )(counts, T, Wg1, bg1.reshape(1, _H), Wg2, bg2.reshape(1, _H),
      Wg3, bg3.reshape(1, _OUT), Wt1, bt1.reshape(1, _H),
      Wt2, bt2.reshape(1, _H), Wt3, bt3.reshape(1, _OUT))


def kernel(group_ids, tech_ids, Eg, Et, Wg1, bg1, Wg2, bg2, Wg3, bg3,
           Wt1, bt1, Wt2, bt2, Wt3, bt3):
    counts = _sc_histogram(group_ids.reshape(-1), tech_ids.reshape(-1))

    m = (jnp.arange(_V) != 0).astype(jnp.float32)[:, None]
    z = jnp.zeros((_V, 1), jnp.float32)
    tg = jnp.concatenate(
        [Eg * m, m] + [z] * (128 - _D - 1), axis=1)
    tt = jnp.concatenate(
        [z] * 64 + [Et * m, m] + [z] * (128 - 64 - _D - 1), axis=1)
    T = jnp.concatenate([
        jnp.pad(tg, ((0, _TOFF - _V), (0, 0))),
        jnp.pad(tt, ((0, _TOFF - _V), (0, 0))),
    ], axis=0)

    return _tc_towers(counts.reshape(_B, _W), T, Wg1, bg1, Wg2, bg2, Wg3, bg3,
                      Wt1, bt1, Wt2, bt2, Wt3, bt3)


# speedup vs baseline: 55.6698x; 1.4201x over previous
import functools

import jax
import jax.numpy as jnp
from jax import lax
from jax.experimental import pallas as pl
from jax.experimental.pallas import tpu as pltpu
from jax.experimental.pallas import tpu_sc as plsc

_B = 4096
_LG = 200
_LT = 400
_V = 1000
_D = 32
_H = 256
_OUT = 64
_W = 2048
_TOFF = 1024

_NC = 2
_NS = 16
_LANES = 16
_NW = _NC * _NS
_RPW = _B // _NW
_R = 16
_NCHUNK = _RPW // _R


def _sc_histogram(gid_flat, tid_flat):
    mesh = plsc.VectorSubcoreMesh(core_axis_name="c", subcore_axis_name="s")

    @functools.partial(
        pl.kernel,
        mesh=mesh,
        compiler_params=pltpu.CompilerParams(needs_layout_passes=False),
        out_type=jax.ShapeDtypeStruct((_B * _W,), jnp.float32),
        scratch_types=[
            pltpu.VMEM((_R * _LG,), jnp.int32),
            pltpu.VMEM((_R * _LT,), jnp.int32),
            pltpu.VMEM((_R * _W,), jnp.float32),
        ],
    )
    def hist_kernel(gid_hbm, tid_hbm, out_hbm, gid_v, tid_v, hist_v):
        wid = lax.axis_index("s") * _NC + lax.axis_index("c")
        base = wid * _RPW
        ones = jnp.ones((_LANES,), jnp.float32)
        lane = lax.iota(jnp.int32, _LANES)
        tailmask = lane >= (16 - _LG % 16)

        def chunk_body(c, carry):
            row0 = base + c * _R
            pltpu.sync_copy(gid_hbm.at[pl.ds(row0 * _LG, _R * _LG)], gid_v)
            pltpu.sync_copy(tid_hbm.at[pl.ds(row0 * _LT, _R * _LT)], tid_v)

            def zero_body(j, carry2):
                for k in range(8):
                    hist_v[pl.ds(j * 128 + k * 16, 16)] = jnp.zeros(
                        (16,), jnp.float32)
                return carry2

            lax.fori_loop(0, (_R * _W) // 128, zero_body, 0)

            def row_body(r, carry2):
                gb = jnp.full((_LANES,), r * _W, jnp.int32)
                tb = jnp.full((_LANES,), r * _W + _TOFF, jnp.int32)
                for cc in range(_LG // 16):
                    g = gid_v[pl.ds(r * _LG + cc * 16, 16)]
                    plsc.addupdate_scatter(hist_v, [g + gb], ones)
                g = gid_v[pl.ds(r * _LG + (_LG - 16), 16)]
                plsc.addupdate_scatter(hist_v, [g + gb], ones, mask=tailmask)
                for cc in range(_LT // 16):
                    t = tid_v[pl.ds(r * _LT + cc * 16, 16)]
                    plsc.addupdate_scatter(hist_v, [t + tb], ones)
                return carry2

            lax.fori_loop(0, _R, row_body, 0)

            pltpu.sync_copy(hist_v, out_hbm.at[pl.ds(row0 * _W, _R * _W)])
            return carry

        lax.fori_loop(0, _NCHUNK, chunk_body, 0)

    return hist_kernel(gid_flat, tid_flat)


def _tc_towers(counts, T, Wg1, bg1, Wg2, bg2, Wg3, bg3,
               Wt1, bt1, Wt2, bt2, Wt3, bt3):
    BB = 256

    def body(c_ref, t_ref, wg1, bg1r, wg2, bg2r, wg3, bg3r,
             wt1, bt1r, wt2, bt2r, wt3, bt3r, out_ref):
        x = jnp.dot(c_ref[...], t_ref[...],
                    preferred_element_type=jnp.float32)
        g = x[:, 0:_D] / jnp.maximum(x[:, _D:_D + 1], 1.0)
        t = x[:, 64:64 + _D] / jnp.maximum(x[:, 64 + _D:64 + _D + 1], 1.0)
        hg = jnp.maximum(jnp.dot(g, wg1[...]) + bg1r[...], 0.0)
        hg = jnp.maximum(jnp.dot(hg, wg2[...]) + bg2r[...], 0.0)
        gv = jnp.dot(hg, wg3[...]) + bg3r[...]
        ht = jnp.maximum(jnp.dot(t, wt1[...]) + bt1r[...], 0.0)
        ht = jnp.maximum(jnp.dot(ht, wt2[...]) + bt2r[...], 0.0)
        tv = jnp.dot(ht, wt3[...]) + bt3r[...]
        out_ref[...] = jax.nn.sigmoid(jnp.sum(gv * tv, axis=1))

    full = lambda shape: pl.BlockSpec(shape, lambda i: (0,) * len(shape))
    return pl.pallas_call(
        body,
        grid=(_B // BB,),
        in_specs=[
            pl.BlockSpec((BB, _W), lambda i: (i, 0)),
            full((_W, 128)),
            full((_D, _H)), full((1, _H)),
            full((_H, _H)), full((1, _H)),
            full((_H, _OUT)), full((1, _OUT)),
            full((_D, _H)), full((1, _H)),
            full((_H, _H)), full((1, _H)),
            full((_H, _OUT)), full((1, _OUT)),
        ],
        out_specs=pl.BlockSpec((BB,), lambda i: (i,)),
        out_shape=jax.ShapeDtypeStruct((_B,), jnp.float32),
    )(counts, T, Wg1, bg1.reshape(1, _H), Wg2, bg2.reshape(1, _H),
      Wg3, bg3.reshape(1, _OUT), Wt1, bt1.reshape(1, _H),
      Wt2, bt2.reshape(1, _H), Wt3, bt3.reshape(1, _OUT))


def kernel(group_ids, tech_ids, Eg, Et, Wg1, bg1, Wg2, bg2, Wg3, bg3,
           Wt1, bt1, Wt2, bt2, Wt3, bt3):
    counts = _sc_histogram(group_ids.reshape(-1), tech_ids.reshape(-1))

    m = (jnp.arange(_V) != 0).astype(jnp.float32)[:, None]
    z = jnp.zeros((_V, 1), jnp.float32)
    tg = jnp.concatenate(
        [Eg * m, m] + [z] * (128 - _D - 1), axis=1)
    tt = jnp.concatenate(
        [z] * 64 + [Et * m, m] + [z] * (128 - 64 - _D - 1), axis=1)
    T = jnp.concatenate([
        jnp.pad(tg, ((0, _TOFF - _V), (0, 0))),
        jnp.pad(tt, ((0, _TOFF - _V), (0, 0))),
    ], axis=0)

    return _tc_towers(counts.reshape(_B, _W), T, Wg1, bg1, Wg2, bg2, Wg3, bg3,
                      Wt1, bt1, Wt2, bt2, Wt3, bt3)


# speedup vs baseline: 65.0819x; 1.1691x over previous
import functools

import jax
import jax.numpy as jnp
from jax import lax
from jax.experimental import pallas as pl
from jax.experimental.pallas import tpu as pltpu
from jax.experimental.pallas import tpu_sc as plsc

_B = 4096
_LG = 200
_LT = 400
_V = 1000
_D = 32
_H = 256
_OUT = 64
_W = 2048
_TOFF = 1024

_NC = 2
_NS = 16
_LANES = 16
_NW = _NC * _NS
_RPW = _B // _NW
_R = 16
_NCHUNK = _RPW // _R


def _sc_histogram(gid_flat, tid_flat):
    mesh = plsc.VectorSubcoreMesh(core_axis_name="c", subcore_axis_name="s")

    @functools.partial(
        pl.kernel,
        mesh=mesh,
        compiler_params=pltpu.CompilerParams(needs_layout_passes=False),
        out_type=jax.ShapeDtypeStruct((_B * _W,), jnp.float32),
        scratch_types=[
            pltpu.VMEM((2, _R * _LG), jnp.int32),
            pltpu.VMEM((2, _R * _LT), jnp.int32),
            pltpu.VMEM((2, _R * _W), jnp.float32),
            pltpu.SemaphoreType.DMA,
            pltpu.SemaphoreType.DMA,
            pltpu.SemaphoreType.DMA,
            pltpu.SemaphoreType.DMA,
            pltpu.SemaphoreType.DMA,
            pltpu.SemaphoreType.DMA,
        ],
    )
    def hist_kernel(gid_hbm, tid_hbm, out_hbm, gid_v, tid_v, hist_v,
                    sg0, sg1, st0, st1, so0, so1):
        sgs = (sg0, sg1)
        sts = (st0, st1)
        sos = (so0, so1)
        wid = lax.axis_index("s") * _NC + lax.axis_index("c")
        base = wid * _RPW
        ones = jnp.ones((_LANES,), jnp.float32)
        lane = lax.iota(jnp.int32, _LANES)
        tailmask = lane >= (16 - _LG % 16)
        zero16 = jnp.zeros((16,), jnp.float32)

        def start_id_load(c, b):
            row0 = base + c * _R
            pltpu.async_copy(
                gid_hbm.at[pl.ds(row0 * _LG, _R * _LG)], gid_v.at[b], sgs[b])
            pltpu.async_copy(
                tid_hbm.at[pl.ds(row0 * _LT, _R * _LT)], tid_v.at[b], sts[b])

        def wait_id(b):
            pltpu.make_async_copy(
                gid_hbm.at[pl.ds(0, _R * _LG)], gid_v.at[b], sgs[b]).wait()
            pltpu.make_async_copy(
                tid_hbm.at[pl.ds(0, _R * _LT)], tid_v.at[b], sts[b]).wait()

        start_id_load(0, 0)
        start_id_load(1, 1)

        def outer_body(o, carry):
            for b in range(2):
                c = o * 2 + b
                row0 = base + c * _R

                @pl.when(c >= 2)
                def _wait_out():
                    pltpu.make_async_copy(
                        hist_v.at[b], out_hbm.at[pl.ds(0, _R * _W)],
                        sos[b]).wait()

                def zero_body(j, carry2):
                    for k in range(16):
                        hist_v[b, pl.ds(j * 256 + k * 16, 16)] = zero16
                    return carry2

                lax.fori_loop(0, (_R * _W) // 256, zero_body, 0)

                wait_id(b)

                bvec = jnp.full((_LANES,), b, jnp.int32)

                def row_body(r, carry2):
                    gb = jnp.full((_LANES,), r * _W, jnp.int32)
                    tb = jnp.full((_LANES,), r * _W + _TOFF, jnp.int32)
                    for cc in range(_LG // 16):
                        g = gid_v[b, pl.ds(r * _LG + cc * 16, 16)]
                        plsc.addupdate_scatter(hist_v, [bvec, g + gb], ones)
                    g = gid_v[b, pl.ds(r * _LG + (_LG - 16), 16)]
                    plsc.addupdate_scatter(
                        hist_v, [bvec, g + gb], ones, mask=tailmask)
                    for cc in range(_LT // 16):
                        t = tid_v[b, pl.ds(r * _LT + cc * 16, 16)]
                        plsc.addupdate_scatter(hist_v, [bvec, t + tb], ones)
                    return carry2

                lax.fori_loop(0, _R, row_body, 0)

                @pl.when(c + 2 < _NCHUNK)
                def _prefetch():
                    start_id_load(c + 2, b)

                pltpu.async_copy(
                    hist_v.at[b], out_hbm.at[pl.ds(row0 * _W, _R * _W)],
                    sos[b])
            return carry

        lax.fori_loop(0, _NCHUNK // 2, outer_body, 0)

        for b in range(2):
            pltpu.make_async_copy(
                hist_v.at[b], out_hbm.at[pl.ds(0, _R * _W)], sos[b]).wait()

    return hist_kernel(gid_flat, tid_flat)


def _tc_towers(counts3, T3, Wg1, bg1, Wg2, bg2, Wg3, bg3,
               Wt1, bt1, Wt2, bt2, Wt3, bt3):
    BB = 256

    def body(c_ref, t_ref, wg1, bg1r, wg2, bg2r, wg3, bg3r,
             wt1, bt1r, wt2, bt2r, wt3, bt3r, out_ref):
        x = jnp.dot(c_ref[:, 0, :], t_ref[0],
                    preferred_element_type=jnp.float32)
        for ct in range(1, 16):
            x = x + jnp.dot(c_ref[:, ct, :], t_ref[ct],
                            preferred_element_type=jnp.float32)
        g = x[:, 0:_D] / jnp.maximum(x[:, _D:_D + 1], 1.0)
        t = x[:, 64:64 + _D] / jnp.maximum(x[:, 64 + _D:64 + _D + 1], 1.0)
        hg = jnp.maximum(jnp.dot(g, wg1[...]) + bg1r[...], 0.0)
        hg = jnp.maximum(jnp.dot(hg, wg2[...]) + bg2r[...], 0.0)
        gv = jnp.dot(hg, wg3[...]) + bg3r[...]
        ht = jnp.maximum(jnp.dot(t, wt1[...]) + bt1r[...], 0.0)
        ht = jnp.maximum(jnp.dot(ht, wt2[...]) + bt2r[...], 0.0)
        tv = jnp.dot(ht, wt3[...]) + bt3r[...]
        out_ref[...] = jax.nn.sigmoid(jnp.sum(gv * tv, axis=1))

    full = lambda shape: pl.BlockSpec(shape, lambda i: (0,) * len(shape))
    return pl.pallas_call(
        body,
        grid=(_B // BB,),
        in_specs=[
            pl.BlockSpec((BB, 16, 128), lambda i: (i, 0, 0)),
            full((16, 128, 128)),
            full((_D, _H)), full((1, _H)),
            full((_H, _H)), full((1, _H)),
            full((_H, _OUT)), full((1, _OUT)),
            full((_D, _H)), full((1, _H)),
            full((_H, _H)), full((1, _H)),
            full((_H, _OUT)), full((1, _OUT)),
        ],
        out_specs=pl.BlockSpec((BB,), lambda i: (i,)),
        out_shape=jax.ShapeDtypeStruct((_B,), jnp.float32),
    )(counts3, T3, Wg1, bg1.reshape(1, _H), Wg2, bg2.reshape(1, _H),
      Wg3, bg3.reshape(1, _OUT), Wt1, bt1.reshape(1, _H),
      Wt2, bt2.reshape(1, _H), Wt3, bt3.reshape(1, _OUT))


def kernel(group_ids, tech_ids, Eg, Et, Wg1, bg1, Wg2, bg2, Wg3, bg3,
           Wt1, bt1, Wt2, bt2, Wt3, bt3):
    counts = _sc_histogram(group_ids.reshape(-1), tech_ids.reshape(-1))

    m = (jnp.arange(_V) != 0).astype(jnp.float32)[:, None]
    z = jnp.zeros((_V, 1), jnp.float32)
    tg = jnp.concatenate(
        [Eg * m, m] + [z] * (128 - _D - 1), axis=1)
    tt = jnp.concatenate(
        [z] * 64 + [Et * m, m] + [z] * (128 - 64 - _D - 1), axis=1)
    T = jnp.concatenate([
        jnp.pad(tg, ((0, _TOFF - _V), (0, 0))),
        jnp.pad(tt, ((0, _TOFF - _V), (0, 0))),
    ], axis=0)

    return _tc_towers(counts.reshape(_B, 16, 128), T.reshape(16, 128, 128),
                      Wg1, bg1, Wg2, bg2, Wg3, bg3,
                      Wt1, bt1, Wt2, bt2, Wt3, bt3)


# speedup vs baseline: 112.4129x; 1.7273x over previous
import functools

import jax
import jax.numpy as jnp
from jax import lax
from jax.experimental import pallas as pl
from jax.experimental.pallas import tpu as pltpu
from jax.experimental.pallas import tpu_sc as plsc

_B = 4096
_LG = 200
_LT = 400
_V = 1000
_D = 32
_H = 256
_OUT = 64
_W = 2048
_TOFF = 1024

_NC = 2
_NS = 16
_LANES = 16
_NW = _NC * _NS
_RPW = _B // _NW
_R = 16
_NCHUNK = _RPW // _R
_GRP = 8


def _sc_histogram(gid, tid):
    mesh = plsc.VectorSubcoreMesh(core_axis_name="c", subcore_axis_name="s")

    @functools.partial(
        pl.kernel,
        mesh=mesh,
        compiler_params=pltpu.CompilerParams(needs_layout_passes=False),
        out_type=jax.ShapeDtypeStruct((_B * _W,), jnp.float32),
        scratch_types=[
            pltpu.VMEM((_R, _LG), jnp.int32),
            pltpu.VMEM((_R, _LG), jnp.int32),
            pltpu.VMEM((_R, _LT), jnp.int32),
            pltpu.VMEM((_R, _LT), jnp.int32),
            pltpu.VMEM((_R * _W,), jnp.float32),
            pltpu.VMEM((_R * _W,), jnp.float32),
            pltpu.SemaphoreType.DMA,
            pltpu.SemaphoreType.DMA,
            pltpu.SemaphoreType.DMA,
            pltpu.SemaphoreType.DMA,
            pltpu.SemaphoreType.DMA,
            pltpu.SemaphoreType.DMA,
        ],
    )
    def hist_kernel(gid_hbm, tid_hbm, out_hbm, gid_v0, gid_v1, tid_v0, tid_v1,
                    hist_v0, hist_v1, sg0, sg1, st0, st1, so0, so1):
        gids = (gid_v0, gid_v1)
        tids = (tid_v0, tid_v1)
        hists = (hist_v0, hist_v1)
        sgs = (sg0, sg1)
        sts = (st0, st1)
        sos = (so0, so1)
        wid = lax.axis_index("s") * _NC + lax.axis_index("c")
        base = wid * _RPW
        ones = jnp.ones((_LANES,), jnp.float32)
        lane = lax.iota(jnp.int32, _LANES)
        tailmask = lane >= (16 - _LG % 16)
        zero16 = jnp.zeros((16,), jnp.float32)

        def start_id_load(c, b):
            row0 = base + c * _R
            pltpu.async_copy(gid_hbm.at[pl.ds(row0, _R)], gids[b], sgs[b])
            pltpu.async_copy(tid_hbm.at[pl.ds(row0, _R)], tids[b], sts[b])

        def wait_id(b):
            pltpu.make_async_copy(
                gid_hbm.at[pl.ds(0, _R)], gids[b], sgs[b]).wait()
            pltpu.make_async_copy(
                tid_hbm.at[pl.ds(0, _R)], tids[b], sts[b]).wait()

        start_id_load(0, 0)
        start_id_load(1, 1)

        def outer_body(o, carry):
            for b in range(2):
                c = o * 2 + b
                row0 = base + c * _R
                hist_b = hists[b]
                gid_b = gids[b]
                tid_b = tids[b]

                @pl.when(c >= 2)
                def _wait_out():
                    pltpu.make_async_copy(
                        hist_b, out_hbm.at[pl.ds(0, _R * _W)], sos[b]).wait()

                def zero_body(j, carry2):
                    for k in range(16):
                        hist_b[pl.ds(j * 256 + k * 16, 16)] = zero16
                    return carry2

                lax.fori_loop(0, (_R * _W) // 256, zero_body, 0)

                wait_id(b)

                def row_body(r, carry2):
                    gb = jnp.full((_LANES,), r * _W, jnp.int32)
                    tb = jnp.full((_LANES,), r * _W + _TOFF, jnp.int32)
                    chunks = (
                        [(cc * 16, gb, gid_b, None)
                         for cc in range(_LG // 16)]
                        + [(_LG - 16, gb, gid_b, tailmask)]
                        + [(cc * 16, tb, tid_b, None)
                           for cc in range(_LT // 16)]
                    )
                    for i in range(0, len(chunks), _GRP):
                        grp = chunks[i:i + _GRP]
                        vals = [ref[r, pl.ds(off, 16)]
                                for off, _, ref, _ in grp]
                        idxs = [v + bb for v, (_, bb, _, _) in zip(vals, grp)]
                        for idx, (_, _, _, msk) in zip(idxs, grp):
                            if msk is None:
                                plsc.addupdate_scatter(hist_b, [idx], ones)
                            else:
                                plsc.addupdate_scatter(
                                    hist_b, [idx], ones, mask=msk)
                    return carry2

                lax.fori_loop(0, _R, row_body, 0)

                @pl.when(c + 2 < _NCHUNK)
                def _prefetch():
                    start_id_load(c + 2, b)

                pltpu.async_copy(
                    hist_b, out_hbm.at[pl.ds(row0 * _W, _R * _W)], sos[b])
            return carry

        lax.fori_loop(0, _NCHUNK // 2, outer_body, 0)

        for b in range(2):
            pltpu.make_async_copy(
                hists[b], out_hbm.at[pl.ds(0, _R * _W)], sos[b]).wait()

    return hist_kernel(gid, tid)


def _tc_towers(counts3, T3, Wg1, bg1, Wg2, bg2, Wg3, bg3,
               Wt1, bt1, Wt2, bt2, Wt3, bt3):
    BB = 512

    def body(c_ref, t_ref, wg1, bg1r, wg2, bg2r, wg3, bg3r,
             wt1, bt1r, wt2, bt2r, wt3, bt3r, out_ref):
        x = jnp.dot(c_ref[:, 0, :], t_ref[0],
                    preferred_element_type=jnp.float32)
        for ct in range(1, 16):
            x = x + jnp.dot(c_ref[:, ct, :], t_ref[ct],
                            preferred_element_type=jnp.float32)
        g = x[:, 0:_D] / jnp.maximum(x[:, _D:_D + 1], 1.0)
        t = x[:, 64:64 + _D] / jnp.maximum(x[:, 64 + _D:64 + _D + 1], 1.0)
        hg = jnp.maximum(jnp.dot(g, wg1[...]) + bg1r[...], 0.0)
        hg = jnp.maximum(jnp.dot(hg, wg2[...]) + bg2r[...], 0.0)
        gv = jnp.dot(hg, wg3[...]) + bg3r[...]
        ht = jnp.maximum(jnp.dot(t, wt1[...]) + bt1r[...], 0.0)
        ht = jnp.maximum(jnp.dot(ht, wt2[...]) + bt2r[...], 0.0)
        tv = jnp.dot(ht, wt3[...]) + bt3r[...]
        out_ref[...] = jax.nn.sigmoid(jnp.sum(gv * tv, axis=1))

    full = lambda shape: pl.BlockSpec(shape, lambda i: (0,) * len(shape))
    return pl.pallas_call(
        body,
        grid=(_B // BB,),
        in_specs=[
            pl.BlockSpec((BB, 16, 128), lambda i: (i, 0, 0)),
            full((16, 128, 128)),
            full((_D, _H)), full((1, _H)),
            full((_H, _H)), full((1, _H)),
            full((_H, _OUT)), full((1, _OUT)),
            full((_D, _H)), full((1, _H)),
            full((_H, _H)), full((1, _H)),
            full((_H, _OUT)), full((1, _OUT)),
        ],
        out_specs=pl.BlockSpec((BB,), lambda i: (i,)),
        out_shape=jax.ShapeDtypeStruct((_B,), jnp.float32),
    )(counts3, T3, Wg1, bg1.reshape(1, _H), Wg2, bg2.reshape(1, _H),
      Wg3, bg3.reshape(1, _OUT), Wt1, bt1.reshape(1, _H),
      Wt2, bt2.reshape(1, _H), Wt3, bt3.reshape(1, _OUT))


def kernel(group_ids, tech_ids, Eg, Et, Wg1, bg1, Wg2, bg2, Wg3, bg3,
           Wt1, bt1, Wt2, bt2, Wt3, bt3):
    counts = _sc_histogram(group_ids, tech_ids)

    m = (jnp.arange(_V) != 0).astype(jnp.float32)[:, None]
    z = jnp.zeros((_V, 1), jnp.float32)
    tg = jnp.concatenate(
        [Eg * m, m] + [z] * (128 - _D - 1), axis=1)
    tt = jnp.concatenate(
        [z] * 64 + [Et * m, m] + [z] * (128 - 64 - _D - 1), axis=1)
    T = jnp.concatenate([
        jnp.pad(tg, ((0, _TOFF - _V), (0, 0))),
        jnp.pad(tt, ((0, _TOFF - _V), (0, 0))),
    ], axis=0)

    return _tc_towers(counts.reshape(_B, 16, 128), T.reshape(16, 128, 128),
                      Wg1, bg1, Wg2, bg2, Wg3, bg3,
                      Wt1, bt1, Wt2, bt2, Wt3, bt3)
